# Initial kernel scaffold; baseline (speedup 1.0000x reference)
import jax
import jax.numpy as jnp
from jax.experimental import pallas as pl


def kernel(x, edge_index, Wp1, bp1, Wl1, bl1, Wr1, Wp2, bp2, Wl2, bl2, Wr2, Wp3, bp3, Wl3, bl3, Wr3):
    raise NotImplementedError("write your pallas kernel here")


# speedup vs baseline: 7.7031x; 7.7031x over previous
import functools

import jax
import jax.numpy as jnp
from jax import lax
from jax.experimental import pallas as pl
from jax.experimental.pallas import tpu as pltpu
from jax.experimental.pallas import tpu_sc as plsc

N = 100000
F = 16
NC, NS, CB = 2, 16, 128
HNP = 50432
TRASH = HNP
AGR = HNP + 128
RPT = AGR // NS
ZB = 40
EPW = 204800
EP = EPW * NS
CHUNK = 32
RPW = EPW // CB
STEPS = RPW // CHUNK
BN = 2000


def _make_segsum():
    mesh = plsc.VectorSubcoreMesh(
        core_axis_name="c", subcore_axis_name="s", num_cores=NC, num_subcores=NS
    )

    @functools.partial(
        pl.kernel,
        out_type=jax.ShapeDtypeStruct((NC, AGR, F), jnp.float32),
        mesh=mesh,
        scratch_types=[
            pltpu.VMEM((CHUNK, CB), jnp.int32),
            pltpu.VMEM((CHUNK, CB), jnp.int32),
            pltpu.VMEM((CHUNK, CB, F), jnp.float32),
            pltpu.VMEM((ZB, F), jnp.float32),
            pltpu.VMEM_SHARED((AGR, F), jnp.float32),
            pltpu.SemaphoreType.DMA,
        ],
        compiler_params=pltpu.CompilerParams(use_tc_tiling_on_sc=False),
    )
    def segsum(h_hbm, src_hbm, dst_hbm, out_hbm, sidx, didx, rows, zbuf, agg, sem):
        c = lax.axis_index("c")
        s = lax.axis_index("s")

        def _zb(i, carry):
            zbuf[i, :] = jnp.zeros((F,), jnp.float32)
            return carry

        lax.fori_loop(0, ZB, _zb, 0)

        def _zi(i, carry):
            pltpu.sync_copy(zbuf, agg.at[pl.ds(s * RPT + i * ZB, ZB)])
            return carry

        lax.fori_loop(0, RPT // ZB, _zi, 0)
        plsc.subcore_barrier()

        base = s * RPW

        def _step(g, carry):
            row0 = base + g * CHUNK
            pltpu.sync_copy(src_hbm.at[pl.ds(row0, CHUNK)], sidx)
            pltpu.sync_copy(dst_hbm.at[c, pl.ds(row0, CHUNK)], didx)

            def _j(j, carry2):
                pltpu.async_copy(h_hbm.at[sidx.at[j]], rows.at[j], sem).wait()
                pltpu.sync_copy(rows.at[j], agg.at[didx.at[j]], add=True)
                return carry2

            lax.fori_loop(0, CHUNK, _j, 0)
            return carry

        lax.fori_loop(0, STEPS, _step, 0)
        plsc.subcore_barrier()
        pltpu.sync_copy(
            agg.at[pl.ds(s * RPT, RPT)], out_hbm.at[c, pl.ds(s * RPT, RPT)]
        )

    return segsum


_SEGSUM = _make_segsum()

_ROW = lambda i: (i, 0)
_FIX = lambda i: (0, 0)


def _tc1_body(x_ref, w_ref, b_ref, h_ref):
    h_ref[...] = jax.nn.relu(
        jnp.dot(x_ref[...], w_ref[...], preferred_element_type=jnp.float32)
        + b_ref[...]
    )


def _tc1(xp, w, b):
    return pl.pallas_call(
        _tc1_body,
        grid=(N // BN,),
        in_specs=[
            pl.BlockSpec((BN, F), _ROW),
            pl.BlockSpec((F, F), _FIX),
            pl.BlockSpec((1, F), _FIX),
        ],
        out_specs=pl.BlockSpec((BN, F), _ROW),
        out_shape=jax.ShapeDtypeStruct((N, F), jnp.float32),
    )(xp, w, b)


def _tc2_body(a0, xr, wl, bl, wr, wp, bp, x2_ref, h2_ref):
    agg = a0[...]
    x2 = jax.nn.relu(
        jnp.dot(agg, wl[...], preferred_element_type=jnp.float32)
        + bl[...]
        + jnp.dot(xr[...], wr[...], preferred_element_type=jnp.float32)
    )
    x2_ref[...] = x2
    h2_ref[...] = jax.nn.relu(
        jnp.dot(x2, wp[...], preferred_element_type=jnp.float32) + bp[...]
    )


def _tc2(a0, xr, wl, bl, wr, wp, bp):
    return pl.pallas_call(
        _tc2_body,
        grid=(N // BN,),
        in_specs=[
            pl.BlockSpec((BN, F), _ROW),
            pl.BlockSpec((BN, F), _ROW),
            pl.BlockSpec((F, F), _FIX),
            pl.BlockSpec((1, F), _FIX),
            pl.BlockSpec((F, F), _FIX),
            pl.BlockSpec((F, F), _FIX),
            pl.BlockSpec((1, F), _FIX),
        ],
        out_specs=[pl.BlockSpec((BN, F), _ROW), pl.BlockSpec((BN, F), _ROW)],
        out_shape=[
            jax.ShapeDtypeStruct((N, F), jnp.float32),
            jax.ShapeDtypeStruct((N, F), jnp.float32),
        ],
    )(a0, xr, wl, bl, wr, wp, bp)


def _tc4_body(a0, xr, wl, bl, wr, out_ref):
    agg = a0[...]
    out_ref[...] = jax.nn.sigmoid(
        jnp.dot(agg, wl[...], preferred_element_type=jnp.float32)
        + bl[...]
        + jnp.dot(xr[...], wr[...], preferred_element_type=jnp.float32)
    )


def _tc4(a0, xr, wl, bl, wr):
    return pl.pallas_call(
        _tc4_body,
        grid=(N // BN,),
        in_specs=[
            pl.BlockSpec((BN, F), _ROW),
            pl.BlockSpec((BN, F), _ROW),
            pl.BlockSpec((F, 1), _FIX),
            pl.BlockSpec((1, 1), _FIX),
            pl.BlockSpec((F, 1), _FIX),
        ],
        out_specs=pl.BlockSpec((BN, 1), _ROW),
        out_shape=jax.ShapeDtypeStruct((N, 1), jnp.float32),
    )(a0, xr, wl, bl, wr)


def kernel(x, edge_index, Wp1, bp1, Wl1, bl1, Wr1, Wp2, bp2, Wl2, bl2, Wr2,
           Wp3, bp3, Wl3, bl3, Wr3):
    f32 = jnp.float32
    xp = jnp.zeros((N, F), f32).at[:, :3].set(x)
    wp1 = jnp.zeros((F, F), f32).at[:3, :3].set(Wp1.T)
    bp1p = jnp.zeros((1, F), f32).at[0, :3].set(bp1)
    wl1 = jnp.zeros((F, F), f32).at[:3, :].set(Wl1.T)
    bl1p = bl1.reshape(1, F)
    wr1 = jnp.zeros((F, F), f32).at[:3, :].set(Wr1.T)
    wp2, bp2p, wl2, bl2p, wr2 = Wp2.T, bp2.reshape(1, F), Wl2.T, bl2.reshape(1, F), Wr2.T
    wp3, bp3p = Wp3.T, bp3.reshape(1, F)
    wl3, bl3p, wr3 = Wl3.T, bl3.reshape(1, 1), Wr3.T

    src = edge_index[0]
    dst = edge_index[1]
    padn = EP - src.shape[0]
    src2 = jnp.concatenate([src, jnp.zeros((padn,), jnp.int32)]).reshape(EP // CB, CB)
    dstp = jnp.concatenate([dst, jnp.full((padn,), 2 * HNP, jnp.int32)])
    d0 = jnp.where(dstp < HNP, dstp, TRASH)
    d1r = dstp - HNP
    d1 = jnp.where(d1r >= 0, jnp.minimum(d1r, TRASH), TRASH)
    dstm = jnp.stack([d0, d1]).reshape(NC, EP // CB, CB)

    def _merge(agg):
        return jnp.concatenate([agg[0, :HNP], agg[1, : N - HNP]], axis=0)

    h1 = _tc1(xp, wp1, bp1p)
    agg1 = _merge(_SEGSUM(h1, src2, dstm))
    x2, h2 = _tc2(agg1, xp, wl1, bl1p, wr1, wp2, bp2p)
    agg2 = _merge(_SEGSUM(h2, src2, dstm))
    x3, h3 = _tc2(agg2, x2, wl2, bl2p, wr2, wp3, bp3p)
    agg3 = _merge(_SEGSUM(h3, src2, dstm))
    out = _tc4(agg3, x3, wl3, bl3p, wr3)
    return out
